# baseline (speedup 1.0000x reference)
import functools

import jax
import jax.numpy as jnp
from jax import lax
from jax.experimental import pallas as pl
from jax.experimental.pallas import tpu as pltpu
from jax.experimental.pallas import tpu_sc as plsc

NC, NS, L = 2, 16, 16
NW = NC * NS
ECH_D = 2000
ECH = 1280
R = 64
CAP_TRIG = 2080
CAP = CAP_TRIG + ECH + 32


def _sc_mesh():
    return plsc.VectorSubcoreMesh(
        core_axis_name="c", subcore_axis_name="s",
        num_cores=NC, num_subcores=NS)


def _wid():
    return lax.axis_index("s") * NC + lax.axis_index("c")


def _degree_partials(cols, n_pad):
    e = cols.shape[0]
    ew = e // NW
    n_chunks = ew // ECH_D

    def body(cols_hbm, out_hbm, hist_v, cbuf):
        wid = _wid()
        zeros = jnp.zeros((L,), jnp.float32)
        ones = jnp.ones((L,), jnp.float32)

        def zero_body(i, _):
            hist_v[pl.ds(i * L, L)] = zeros
            return 0
        lax.fori_loop(0, n_pad // L, zero_body, 0)

        base = wid * ew
        for ch in range(n_chunks):
            pltpu.sync_copy(cols_hbm.at[pl.ds(base + ch * ECH_D, ECH_D)],
                            cbuf)

            def scan_body(v, _):
                c = cbuf[pl.ds(v * L, L)]
                plsc.addupdate_scatter(hist_v, [c], ones)
                return 0
            lax.fori_loop(0, ECH_D // L, scan_body, 0)

        pltpu.sync_copy(hist_v, out_hbm.at[wid])

    f = pl.kernel(
        body,
        out_type=jax.ShapeDtypeStruct((NW, n_pad), jnp.float32),
        mesh=_sc_mesh(),
        compiler_params=pltpu.CompilerParams(needs_layout_passes=False),
        scratch_types=[
            pltpu.VMEM((n_pad,), jnp.float32),
            pltpu.VMEM((ECH_D,), jnp.int32),
        ],
    )
    return f(cols)


def _tc_dis(parts, n_real):
    n_pad = parts.shape[1]

    def body(p_ref, o_ref):
        s = jnp.sum(p_ref[...], axis=0) + 1.0
        idx = lax.broadcasted_iota(jnp.int32, (1, n_pad), 1)
        o_ref[...] = jnp.where(idx < n_real, lax.rsqrt(s)[None, :], 0.0)

    return pl.pallas_call(
        body,
        out_shape=jax.ShapeDtypeStruct((1, n_pad), jnp.float32),
    )(parts)


def _tc_scaled_mm(x_pad, w, dis, t=1024):
    b_sz, n_pad, f = x_pad.shape
    grid = (n_pad // t, b_sz)

    def body(x_ref, w_ref, d_ref, o_ref):
        y = jnp.dot(x_ref[0], w_ref[...], preferred_element_type=jnp.float32)
        o_ref[...] = y * d_ref[0][:, None]

    return pl.pallas_call(
        body,
        grid=grid,
        in_specs=[
            pl.BlockSpec((1, t, f), lambda i, b: (b, i, 0)),
            pl.BlockSpec((f, f), lambda i, b: (0, 0)),
            pl.BlockSpec((1, t), lambda i, b: (0, i)),
        ],
        out_specs=pl.BlockSpec((t, f), lambda i, b: (i, b)),
        out_shape=jax.ShapeDtypeStruct((n_pad, b_sz * f), jnp.float32),
    )(x_pad, w, dis)


def _tc_mid(acc, dis, b1, w2, t=1024):
    n_pad, bf = acc.shape
    f = w2.shape[0]
    grid = (n_pad // t, bf // f)

    def body(a_ref, d_ref, b_ref, w_ref, o_ref):
        d = d_ref[0][:, None]
        h = jnp.maximum(a_ref[...] * d + b_ref[...], 0.0)
        o_ref[...] = jnp.dot(h, w_ref[...],
                             preferred_element_type=jnp.float32) * d

    return pl.pallas_call(
        body,
        grid=grid,
        in_specs=[
            pl.BlockSpec((t, f), lambda i, b: (i, b)),
            pl.BlockSpec((1, t), lambda i, b: (0, i)),
            pl.BlockSpec((1, f), lambda i, b: (0, 0)),
            pl.BlockSpec((f, f), lambda i, b: (0, 0)),
        ],
        out_specs=pl.BlockSpec((t, f), lambda i, b: (i, b)),
        out_shape=jax.ShapeDtypeStruct((n_pad, bf), jnp.float32),
    )(acc, dis, b1, w2)


def _tc_final(acc, dis, b2, t=1024):
    n_pad, bf = acc.shape
    f = b2.shape[1]
    b_sz = bf // f
    grid = (n_pad // t, b_sz)

    def body(a_ref, d_ref, b_ref, o_ref):
        o_ref[0] = a_ref[...] * d_ref[0][:, None] + b_ref[...]

    return pl.pallas_call(
        body,
        grid=grid,
        in_specs=[
            pl.BlockSpec((t, f), lambda i, b: (i, b)),
            pl.BlockSpec((1, t), lambda i, b: (0, i)),
            pl.BlockSpec((1, f), lambda i, b: (0, 0)),
        ],
        out_specs=pl.BlockSpec((1, t, f), lambda i, b: (b, i, 0)),
        out_shape=jax.ShapeDtypeStruct((b_sz, n_pad, f), jnp.float32),
    )(acc, dis, b2)


def _sc_spmm(y, rows, cols):
    n_pad, bf = y.shape
    e = rows.shape[0]
    n_passes = n_pad // (NW * R)
    assert n_pad % (NW * R) == 0 and e % ECH == 0

    n_chunks = e // ECH

    def body(y_hbm, rows_hbm, cols_hbm, out_hbm,
             acc, staged, rbuf, cbuf, rowbuf, lcolbuf, pcbuf, gsem, esem):
        wid = _wid()
        lane = lax.iota(jnp.int32, L)
        pad_row = jnp.full((L,), n_pad - 1, jnp.int32)
        pad_lcol = jnp.zeros((L,), jnp.int32)

        def issue_chunk(ec, par):
            pltpu.make_async_copy(rows_hbm.at[pl.ds(ec * ECH, ECH)],
                                  rbuf.at[pl.ds(par * ECH, ECH)],
                                  esem.at[par]).start()
            pltpu.make_async_copy(cols_hbm.at[pl.ds(ec * ECH, ECH)],
                                  cbuf.at[pl.ds(par * ECH, ECH)],
                                  esem.at[par]).start()

        def wait_chunk(par):
            pltpu.make_async_copy(rows_hbm.at[pl.ds(0, ECH)],
                                  rbuf.at[pl.ds(par * ECH, ECH)],
                                  esem.at[par]).wait()
            pltpu.make_async_copy(cols_hbm.at[pl.ds(0, ECH)],
                                  cbuf.at[pl.ds(par * ECH, ECH)],
                                  esem.at[par]).wait()

        def issue_gather(g, par):
            rvec = rowbuf[pl.ds(g * L, L)]
            pltpu.make_async_copy(y_hbm.at[rvec],
                                  staged.at[pl.ds(par * L, L)],
                                  gsem.at[par]).start()

        def wait_gather(g, par):
            rvec = rowbuf[pl.ds(g * L, L)]
            pltpu.make_async_copy(y_hbm.at[rvec],
                                  staged.at[pl.ds(par * L, L)],
                                  gsem.at[par]).wait()

        def drain_all(cnt):
            rowbuf[pl.ds(cnt, L)] = pad_row
            lcolbuf[pl.ds(cnt, L)] = pad_lcol
            ng = (cnt + (L - 1)) // L

            @pl.when(ng > 0)
            def _():
                issue_gather(0, 0)

            @pl.when(ng > 1)
            def _():
                issue_gather(1, 1)

            def drain_body(g, _):
                gpar = lax.rem(g, 3)
                wait_gather(g, gpar)

                @pl.when(g + 2 < ng)
                def _():
                    issue_gather(g + 2, lax.rem(g + 2, 3))

                lvec = lcolbuf[pl.ds(g * L, L)]

                def edge_body(k, _):
                    lc = jnp.sum(jnp.where(lane == k, lvec, 0))
                    srow = gpar * L + k
                    for j in range(bf // L):
                        xv = staged[srow, pl.ds(j * L, L)]
                        plsc.addupdate(acc.at[lc, pl.ds(j * L, L)], xv)
                    return 0

                lax.fori_loop(0, L, edge_body, 0)
                return 0

            lax.fori_loop(0, ng, drain_body, 0)
            return jnp.int32(0)

        for p in range(n_passes):
            lo = (p * NW + wid) * R
            pltpu.sync_copy(y_hbm.at[pl.ds(lo, R)], acc.at[pl.ds(0, R)])
            issue_chunk(0, 0)

            def chunk_body(ec, cnt):
                cpar = lax.rem(ec, 2)
                wait_chunk(cpar)

                @pl.when(ec + 1 < n_chunks)
                def _():
                    issue_chunk(ec + 1, 1 - cpar)

                base = cpar * ECH

                def count_body(v, run):
                    c = cbuf[pl.ds(base + v * L, L)]
                    m = (c >= lo) & (c < lo + R)
                    pcbuf[pl.ds(v * L, L)] = run
                    return run + plsc.all_reduce_population_count(m)

                run = lax.fori_loop(0, ECH // L, count_body,
                                    jnp.broadcast_to(cnt, (L,)))

                def place_body(v, _):
                    c = cbuf[pl.ds(base + v * L, L)]
                    r = rbuf[pl.ds(base + v * L, L)]
                    m = (c >= lo) & (c < lo + R)
                    off_s = pcbuf[pl.ds(v * L, L)]
                    off = jnp.sum(jnp.where(lane == 0, off_s, 0))
                    plsc.store_compressed(rowbuf.at[pl.ds(off, L)], r, mask=m)
                    plsc.store_compressed(lcolbuf.at[pl.ds(off, L)],
                                          c - lo, mask=m)
                    return 0

                lax.fori_loop(0, ECH // L, place_body, 0)
                cnt = jnp.sum(jnp.where(lane == 0, run, 0))
                return lax.cond(cnt > CAP_TRIG, drain_all, lambda x: x, cnt)

            cnt = lax.fori_loop(0, n_chunks, chunk_body, jnp.int32(0))
            drain_all(cnt)
            pltpu.sync_copy(acc.at[pl.ds(0, R)], out_hbm.at[pl.ds(lo, R)])

    f = pl.kernel(
        body,
        out_type=jax.ShapeDtypeStruct((n_pad, bf), jnp.float32),
        mesh=_sc_mesh(),
        compiler_params=pltpu.CompilerParams(needs_layout_passes=False),
        scratch_types=[
            pltpu.VMEM((R, bf), jnp.float32),
            pltpu.VMEM((3 * L, bf), jnp.float32),
            pltpu.VMEM((2 * ECH,), jnp.int32),
            pltpu.VMEM((2 * ECH,), jnp.int32),
            pltpu.VMEM((CAP,), jnp.int32),
            pltpu.VMEM((CAP,), jnp.int32),
            pltpu.VMEM((ECH,), jnp.int32),
            pltpu.SemaphoreType.DMA((3,)),
            pltpu.SemaphoreType.DMA((2,)),
        ],
    )
    return f(y, rows, cols)


def kernel(x, edge_index, W1, b1, W2, b2):
    b_sz, n, f = x.shape
    n_pad = NW * R * -(-n // (NW * R))

    rows = edge_index[0].astype(jnp.int32)
    cols = edge_index[1].astype(jnp.int32)
    e = rows.shape[0]
    e_pad = NW * ECH_D * -(-e // (NW * ECH_D))
    if e_pad != e:
        rows = jnp.concatenate(
            [rows, jnp.zeros((e_pad - e,), jnp.int32)])
        cols = jnp.concatenate(
            [cols, jnp.full((e_pad - e,), n, jnp.int32)])

    x_pad = jnp.pad(x, ((0, 0), (0, n_pad - n), (0, 0)))
    b1r = b1.reshape(1, -1)
    b2r = b2.reshape(1, -1)

    parts = _degree_partials(cols, n_pad)
    dis = _tc_dis(parts, n)

    y1 = _tc_scaled_mm(x_pad, W1, dis)
    acc1 = _sc_spmm(y1, rows, cols)
    y2 = _tc_mid(acc1, dis, b1r, W2)
    acc2 = _sc_spmm(y2, rows, cols)
    out = _tc_final(acc2, dis, b2r)
    return out[:, :n, :]


# speedup vs baseline: 1.1124x; 1.1124x over previous
import functools

import jax
import jax.numpy as jnp
from jax import lax
from jax.experimental import pallas as pl
from jax.experimental.pallas import tpu as pltpu
from jax.experimental.pallas import tpu_sc as plsc

NC, NS, L = 2, 16, 16
NW = NC * NS
ECH_D = 2000
ECH = 1280
R = 80
CAP_TRIG = 2080
CAP = CAP_TRIG + ECH + 32


def _sc_mesh():
    return plsc.VectorSubcoreMesh(
        core_axis_name="c", subcore_axis_name="s",
        num_cores=NC, num_subcores=NS)


def _wid():
    return lax.axis_index("s") * NC + lax.axis_index("c")


def _degree_partials(cols, n_pad):
    e = cols.shape[0]
    ew = e // NW
    n_chunks = ew // ECH_D

    def body(cols_hbm, out_hbm, hist_v, cbuf):
        wid = _wid()
        zeros = jnp.zeros((L,), jnp.float32)
        ones = jnp.ones((L,), jnp.float32)

        def zero_body(i, _):
            hist_v[pl.ds(i * L, L)] = zeros
            return 0
        lax.fori_loop(0, n_pad // L, zero_body, 0)

        base = wid * ew
        for ch in range(n_chunks):
            pltpu.sync_copy(cols_hbm.at[pl.ds(base + ch * ECH_D, ECH_D)],
                            cbuf)

            def scan_body(v, _):
                c = cbuf[pl.ds(v * L, L)]
                plsc.addupdate_scatter(hist_v, [c], ones)
                return 0
            lax.fori_loop(0, ECH_D // L, scan_body, 0)

        pltpu.sync_copy(hist_v, out_hbm.at[wid])

    f = pl.kernel(
        body,
        out_type=jax.ShapeDtypeStruct((NW, n_pad), jnp.float32),
        mesh=_sc_mesh(),
        compiler_params=pltpu.CompilerParams(needs_layout_passes=False),
        scratch_types=[
            pltpu.VMEM((n_pad,), jnp.float32),
            pltpu.VMEM((ECH_D,), jnp.int32),
        ],
    )
    return f(cols)


def _tc_dis(parts, n_real):
    n_pad = parts.shape[1]

    def body(p_ref, o_ref):
        s = jnp.sum(p_ref[...], axis=0) + 1.0
        idx = lax.broadcasted_iota(jnp.int32, (1, n_pad), 1)
        o_ref[...] = jnp.where(idx < n_real, lax.rsqrt(s)[None, :], 0.0)

    return pl.pallas_call(
        body,
        out_shape=jax.ShapeDtypeStruct((1, n_pad), jnp.float32),
    )(parts)


def _tc_scaled_mm(x_pad, w, dis, t=1024):
    b_sz, n_pad, f = x_pad.shape
    grid = (n_pad // t, b_sz)

    def body(x_ref, w_ref, d_ref, o_ref):
        y = jnp.dot(x_ref[0], w_ref[...], preferred_element_type=jnp.float32)
        o_ref[...] = y * d_ref[0][:, None]

    return pl.pallas_call(
        body,
        grid=grid,
        in_specs=[
            pl.BlockSpec((1, t, f), lambda i, b: (b, i, 0)),
            pl.BlockSpec((f, f), lambda i, b: (0, 0)),
            pl.BlockSpec((1, t), lambda i, b: (0, i)),
        ],
        out_specs=pl.BlockSpec((t, f), lambda i, b: (i, b)),
        out_shape=jax.ShapeDtypeStruct((n_pad, b_sz * f), jnp.float32),
    )(x_pad, w, dis)


def _tc_mid(acc, dis, b1, w2, t=1024):
    n_pad, bf = acc.shape
    f = w2.shape[0]
    grid = (n_pad // t, bf // f)

    def body(a_ref, d_ref, b_ref, w_ref, o_ref):
        d = d_ref[0][:, None]
        h = jnp.maximum(a_ref[...] * d + b_ref[...], 0.0)
        o_ref[...] = jnp.dot(h, w_ref[...],
                             preferred_element_type=jnp.float32) * d

    return pl.pallas_call(
        body,
        grid=grid,
        in_specs=[
            pl.BlockSpec((t, f), lambda i, b: (i, b)),
            pl.BlockSpec((1, t), lambda i, b: (0, i)),
            pl.BlockSpec((1, f), lambda i, b: (0, 0)),
            pl.BlockSpec((f, f), lambda i, b: (0, 0)),
        ],
        out_specs=pl.BlockSpec((t, f), lambda i, b: (i, b)),
        out_shape=jax.ShapeDtypeStruct((n_pad, bf), jnp.float32),
    )(acc, dis, b1, w2)


def _tc_final(acc, dis, b2, t=1024):
    n_pad, bf = acc.shape
    f = b2.shape[1]
    b_sz = bf // f
    grid = (n_pad // t, b_sz)

    def body(a_ref, d_ref, b_ref, o_ref):
        o_ref[0] = a_ref[...] * d_ref[0][:, None] + b_ref[...]

    return pl.pallas_call(
        body,
        grid=grid,
        in_specs=[
            pl.BlockSpec((t, f), lambda i, b: (i, b)),
            pl.BlockSpec((1, t), lambda i, b: (0, i)),
            pl.BlockSpec((1, f), lambda i, b: (0, 0)),
        ],
        out_specs=pl.BlockSpec((1, t, f), lambda i, b: (b, i, 0)),
        out_shape=jax.ShapeDtypeStruct((b_sz, n_pad, f), jnp.float32),
    )(acc, dis, b2)


def _sc_spmm(y, rows, cols):
    n_pad, bf = y.shape
    e = rows.shape[0]
    n_passes = n_pad // (NW * R)
    assert n_pad % (NW * R) == 0 and e % ECH == 0

    n_chunks = e // ECH

    def body(y_hbm, rows_hbm, cols_hbm, out_hbm,
             acc, staged, rbuf, cbuf, rowbuf, lcolbuf, pcbuf, gsem, esem):
        wid = _wid()
        lane = lax.iota(jnp.int32, L)
        pad_row = jnp.full((L,), n_pad - 1, jnp.int32)
        pad_lcol = jnp.zeros((L,), jnp.int32)

        def issue_chunk(ec, par):
            pltpu.make_async_copy(rows_hbm.at[pl.ds(ec * ECH, ECH)],
                                  rbuf.at[pl.ds(par * ECH, ECH)],
                                  esem.at[par]).start()
            pltpu.make_async_copy(cols_hbm.at[pl.ds(ec * ECH, ECH)],
                                  cbuf.at[pl.ds(par * ECH, ECH)],
                                  esem.at[par]).start()

        def wait_chunk(par):
            pltpu.make_async_copy(rows_hbm.at[pl.ds(0, ECH)],
                                  rbuf.at[pl.ds(par * ECH, ECH)],
                                  esem.at[par]).wait()
            pltpu.make_async_copy(cols_hbm.at[pl.ds(0, ECH)],
                                  cbuf.at[pl.ds(par * ECH, ECH)],
                                  esem.at[par]).wait()

        def issue_gather(g, par):
            rvec = rowbuf[pl.ds(g * L, L)]
            pltpu.make_async_copy(y_hbm.at[rvec],
                                  staged.at[pl.ds(par * L, L)],
                                  gsem.at[par]).start()

        def wait_gather(g, par):
            rvec = rowbuf[pl.ds(g * L, L)]
            pltpu.make_async_copy(y_hbm.at[rvec],
                                  staged.at[pl.ds(par * L, L)],
                                  gsem.at[par]).wait()

        def drain_all(cnt):
            rowbuf[pl.ds(cnt, L)] = pad_row
            lcolbuf[pl.ds(cnt, L)] = pad_lcol
            ng = (cnt + (L - 1)) // L

            @pl.when(ng > 0)
            def _():
                issue_gather(0, 0)

            def drain_body(g, _):
                gpar = lax.rem(g, 2)
                wait_gather(g, gpar)

                @pl.when(g + 1 < ng)
                def _():
                    issue_gather(g + 1, 1 - gpar)

                lvec = lcolbuf[pl.ds(g * L, L)]

                def edge_body(k, _):
                    lc = jnp.sum(jnp.where(lane == k, lvec, 0))
                    srow = gpar * L + k
                    for j in range(bf // L):
                        xv = staged[srow, pl.ds(j * L, L)]
                        plsc.addupdate(acc.at[lc, pl.ds(j * L, L)], xv)
                    return 0

                lax.fori_loop(0, L, edge_body, 0)
                return 0

            lax.fori_loop(0, ng, drain_body, 0)
            return jnp.int32(0)

        for p in range(n_passes):
            lo = (p * NW + wid) * R
            pltpu.sync_copy(y_hbm.at[pl.ds(lo, R)], acc.at[pl.ds(0, R)])
            issue_chunk(0, 0)

            def chunk_body(ec, cnt):
                cpar = lax.rem(ec, 2)
                wait_chunk(cpar)

                @pl.when(ec + 1 < n_chunks)
                def _():
                    issue_chunk(ec + 1, 1 - cpar)

                base = cpar * ECH

                def count_body(v, run):
                    c = cbuf[pl.ds(base + v * L, L)]
                    m = (c >= lo) & (c < lo + R)
                    pcbuf[pl.ds(v * L, L)] = run
                    return run + plsc.all_reduce_population_count(m)

                run = lax.fori_loop(0, ECH // L, count_body,
                                    jnp.broadcast_to(cnt, (L,)))

                def place_body(v, _):
                    c = cbuf[pl.ds(base + v * L, L)]
                    r = rbuf[pl.ds(base + v * L, L)]
                    m = (c >= lo) & (c < lo + R)
                    off_s = pcbuf[pl.ds(v * L, L)]
                    off = jnp.sum(jnp.where(lane == 0, off_s, 0))
                    plsc.store_compressed(rowbuf.at[pl.ds(off, L)], r, mask=m)
                    plsc.store_compressed(lcolbuf.at[pl.ds(off, L)],
                                          c - lo, mask=m)
                    return 0

                lax.fori_loop(0, ECH // L, place_body, 0)
                cnt = jnp.sum(jnp.where(lane == 0, run, 0))
                return lax.cond(cnt > CAP_TRIG, drain_all, lambda x: x, cnt)

            cnt = lax.fori_loop(0, n_chunks, chunk_body, jnp.int32(0))
            drain_all(cnt)
            pltpu.sync_copy(acc.at[pl.ds(0, R)], out_hbm.at[pl.ds(lo, R)])

    f = pl.kernel(
        body,
        out_type=jax.ShapeDtypeStruct((n_pad, bf), jnp.float32),
        mesh=_sc_mesh(),
        compiler_params=pltpu.CompilerParams(needs_layout_passes=False),
        scratch_types=[
            pltpu.VMEM((R, bf), jnp.float32),
            pltpu.VMEM((2 * L, bf), jnp.float32),
            pltpu.VMEM((2 * ECH,), jnp.int32),
            pltpu.VMEM((2 * ECH,), jnp.int32),
            pltpu.VMEM((CAP,), jnp.int32),
            pltpu.VMEM((CAP,), jnp.int32),
            pltpu.VMEM((ECH,), jnp.int32),
            pltpu.SemaphoreType.DMA((2,)),
            pltpu.SemaphoreType.DMA((2,)),
        ],
    )
    return f(y, rows, cols)


def kernel(x, edge_index, W1, b1, W2, b2):
    b_sz, n, f = x.shape
    n_pad = NW * R * -(-n // (NW * R))

    rows = edge_index[0].astype(jnp.int32)
    cols = edge_index[1].astype(jnp.int32)
    e = rows.shape[0]
    e_pad = NW * ECH_D * -(-e // (NW * ECH_D))
    if e_pad != e:
        rows = jnp.concatenate(
            [rows, jnp.zeros((e_pad - e,), jnp.int32)])
        cols = jnp.concatenate(
            [cols, jnp.full((e_pad - e,), n, jnp.int32)])

    x_pad = jnp.pad(x, ((0, 0), (0, n_pad - n), (0, 0)))
    b1r = b1.reshape(1, -1)
    b2r = b2.reshape(1, -1)

    parts = _degree_partials(cols, n_pad)
    dis = _tc_dis(parts, n)

    y1 = _tc_scaled_mm(x_pad, W1, dis)
    acc1 = _sc_spmm(y1, rows, cols)
    y2 = _tc_mid(acc1, dis, b1r, W2)
    acc2 = _sc_spmm(y2, rows, cols)
    out = _tc_final(acc2, dis, b2r)
    return out[:, :n, :]


# speedup vs baseline: 1.1139x; 1.0014x over previous
import functools

import jax
import jax.numpy as jnp
from jax import lax
from jax.experimental import pallas as pl
from jax.experimental.pallas import tpu as pltpu
from jax.experimental.pallas import tpu_sc as plsc

NC, NS, L = 2, 16, 16
NW = NC * NS
ECH_D = 2000
ECH = 1600
R = 80
CAP_TRIG = 2080
CAP = CAP_TRIG + ECH + 32


def _sc_mesh():
    return plsc.VectorSubcoreMesh(
        core_axis_name="c", subcore_axis_name="s",
        num_cores=NC, num_subcores=NS)


def _wid():
    return lax.axis_index("s") * NC + lax.axis_index("c")


def _degree_partials(cols, n_pad):
    e = cols.shape[0]
    ew = e // NW
    n_chunks = ew // ECH_D

    def body(cols_hbm, out_hbm, hist_v, cbuf):
        wid = _wid()
        zeros = jnp.zeros((L,), jnp.float32)
        ones = jnp.ones((L,), jnp.float32)

        def zero_body(i, _):
            hist_v[pl.ds(i * L, L)] = zeros
            return 0
        lax.fori_loop(0, n_pad // L, zero_body, 0)

        base = wid * ew
        for ch in range(n_chunks):
            pltpu.sync_copy(cols_hbm.at[pl.ds(base + ch * ECH_D, ECH_D)],
                            cbuf)

            def scan_body(v, _):
                c = cbuf[pl.ds(v * L, L)]
                plsc.addupdate_scatter(hist_v, [c], ones)
                return 0
            lax.fori_loop(0, ECH_D // L, scan_body, 0)

        pltpu.sync_copy(hist_v, out_hbm.at[wid])

    f = pl.kernel(
        body,
        out_type=jax.ShapeDtypeStruct((NW, n_pad), jnp.float32),
        mesh=_sc_mesh(),
        compiler_params=pltpu.CompilerParams(needs_layout_passes=False),
        scratch_types=[
            pltpu.VMEM((n_pad,), jnp.float32),
            pltpu.VMEM((ECH_D,), jnp.int32),
        ],
    )
    return f(cols)


def _tc_dis(parts, n_real):
    n_pad = parts.shape[1]

    def body(p_ref, o_ref):
        s = jnp.sum(p_ref[...], axis=0) + 1.0
        idx = lax.broadcasted_iota(jnp.int32, (1, n_pad), 1)
        o_ref[...] = jnp.where(idx < n_real, lax.rsqrt(s)[None, :], 0.0)

    return pl.pallas_call(
        body,
        out_shape=jax.ShapeDtypeStruct((1, n_pad), jnp.float32),
    )(parts)


def _tc_scaled_mm(x_pad, w, dis, t=1024):
    b_sz, n_pad, f = x_pad.shape
    grid = (n_pad // t, b_sz)

    def body(x_ref, w_ref, d_ref, o_ref):
        y = jnp.dot(x_ref[0], w_ref[...], preferred_element_type=jnp.float32)
        o_ref[...] = y * d_ref[0][:, None]

    return pl.pallas_call(
        body,
        grid=grid,
        in_specs=[
            pl.BlockSpec((1, t, f), lambda i, b: (b, i, 0)),
            pl.BlockSpec((f, f), lambda i, b: (0, 0)),
            pl.BlockSpec((1, t), lambda i, b: (0, i)),
        ],
        out_specs=pl.BlockSpec((t, f), lambda i, b: (i, b)),
        out_shape=jax.ShapeDtypeStruct((n_pad, b_sz * f), jnp.float32),
    )(x_pad, w, dis)


def _tc_mid(acc, dis, b1, w2, t=1024):
    n_pad, bf = acc.shape
    f = w2.shape[0]
    grid = (n_pad // t, bf // f)

    def body(a_ref, d_ref, b_ref, w_ref, o_ref):
        d = d_ref[0][:, None]
        h = jnp.maximum(a_ref[...] * d + b_ref[...], 0.0)
        o_ref[...] = jnp.dot(h, w_ref[...],
                             preferred_element_type=jnp.float32) * d

    return pl.pallas_call(
        body,
        grid=grid,
        in_specs=[
            pl.BlockSpec((t, f), lambda i, b: (i, b)),
            pl.BlockSpec((1, t), lambda i, b: (0, i)),
            pl.BlockSpec((1, f), lambda i, b: (0, 0)),
            pl.BlockSpec((f, f), lambda i, b: (0, 0)),
        ],
        out_specs=pl.BlockSpec((t, f), lambda i, b: (i, b)),
        out_shape=jax.ShapeDtypeStruct((n_pad, bf), jnp.float32),
    )(acc, dis, b1, w2)


def _tc_final(acc, dis, b2, t=1024):
    n_pad, bf = acc.shape
    f = b2.shape[1]
    b_sz = bf // f
    grid = (n_pad // t, b_sz)

    def body(a_ref, d_ref, b_ref, o_ref):
        o_ref[0] = a_ref[...] * d_ref[0][:, None] + b_ref[...]

    return pl.pallas_call(
        body,
        grid=grid,
        in_specs=[
            pl.BlockSpec((t, f), lambda i, b: (i, b)),
            pl.BlockSpec((1, t), lambda i, b: (0, i)),
            pl.BlockSpec((1, f), lambda i, b: (0, 0)),
        ],
        out_specs=pl.BlockSpec((1, t, f), lambda i, b: (b, i, 0)),
        out_shape=jax.ShapeDtypeStruct((b_sz, n_pad, f), jnp.float32),
    )(acc, dis, b2)


def _sc_spmm(y, rows, cols):
    n_pad, bf = y.shape
    e = rows.shape[0]
    n_passes = n_pad // (NW * R)
    assert n_pad % (NW * R) == 0 and e % ECH == 0

    n_chunks = e // ECH

    def body(y_hbm, rows_hbm, cols_hbm, out_hbm,
             acc, staged, rbuf, cbuf, rowbuf, lcolbuf, pcbuf, gsem, esem):
        wid = _wid()
        lane = lax.iota(jnp.int32, L)
        pad_row = jnp.full((L,), n_pad - 1, jnp.int32)
        pad_lcol = jnp.zeros((L,), jnp.int32)

        def issue_chunk(ec, par):
            pltpu.make_async_copy(rows_hbm.at[pl.ds(ec * ECH, ECH)],
                                  rbuf.at[pl.ds(par * ECH, ECH)],
                                  esem.at[par]).start()
            pltpu.make_async_copy(cols_hbm.at[pl.ds(ec * ECH, ECH)],
                                  cbuf.at[pl.ds(par * ECH, ECH)],
                                  esem.at[par]).start()

        def wait_chunk(par):
            pltpu.make_async_copy(rows_hbm.at[pl.ds(0, ECH)],
                                  rbuf.at[pl.ds(par * ECH, ECH)],
                                  esem.at[par]).wait()
            pltpu.make_async_copy(cols_hbm.at[pl.ds(0, ECH)],
                                  cbuf.at[pl.ds(par * ECH, ECH)],
                                  esem.at[par]).wait()

        def issue_gather(g, par):
            rvec = rowbuf[pl.ds(g * L, L)]
            pltpu.make_async_copy(y_hbm.at[rvec],
                                  staged.at[pl.ds(par * L, L)],
                                  gsem.at[par]).start()

        def wait_gather(g, par):
            rvec = rowbuf[pl.ds(g * L, L)]
            pltpu.make_async_copy(y_hbm.at[rvec],
                                  staged.at[pl.ds(par * L, L)],
                                  gsem.at[par]).wait()

        def drain_all(cnt):
            rowbuf[pl.ds(cnt, L)] = pad_row
            lcolbuf[pl.ds(cnt, L)] = pad_lcol
            ng = (cnt + (L - 1)) // L

            @pl.when(ng > 0)
            def _():
                issue_gather(0, 0)

            def drain_body(g, _):
                gpar = lax.rem(g, 2)
                wait_gather(g, gpar)

                @pl.when(g + 1 < ng)
                def _():
                    issue_gather(g + 1, 1 - gpar)

                lvec = lcolbuf[pl.ds(g * L, L)]

                def edge_body(k, _):
                    lc = jnp.sum(jnp.where(lane == k, lvec, 0))
                    srow = gpar * L + k
                    for j in range(bf // L):
                        xv = staged[srow, pl.ds(j * L, L)]
                        plsc.addupdate(acc.at[lc, pl.ds(j * L, L)], xv)
                    return 0

                lax.fori_loop(0, L, edge_body, 0)
                return 0

            lax.fori_loop(0, ng, drain_body, 0)
            return jnp.int32(0)

        for p in range(n_passes):
            lo = (p * NW + wid) * R
            pltpu.sync_copy(y_hbm.at[pl.ds(lo, R)], acc.at[pl.ds(0, R)])
            issue_chunk(0, 0)

            def chunk_body(ec, cnt):
                cpar = lax.rem(ec, 2)
                wait_chunk(cpar)

                @pl.when(ec + 1 < n_chunks)
                def _():
                    issue_chunk(ec + 1, 1 - cpar)

                base = cpar * ECH

                def count_body(v, run):
                    c = cbuf[pl.ds(base + v * L, L)]
                    m = (c >= lo) & (c < lo + R)
                    pcbuf[pl.ds(v * L, L)] = run
                    return run + plsc.all_reduce_population_count(m)

                run = lax.fori_loop(0, ECH // L, count_body,
                                    jnp.broadcast_to(cnt, (L,)))

                def place_body(v, _):
                    c = cbuf[pl.ds(base + v * L, L)]
                    r = rbuf[pl.ds(base + v * L, L)]
                    m = (c >= lo) & (c < lo + R)
                    off_s = pcbuf[pl.ds(v * L, L)]
                    off = jnp.sum(jnp.where(lane == 0, off_s, 0))
                    plsc.store_compressed(rowbuf.at[pl.ds(off, L)], r, mask=m)
                    plsc.store_compressed(lcolbuf.at[pl.ds(off, L)],
                                          c - lo, mask=m)
                    return 0

                lax.fori_loop(0, ECH // L, place_body, 0)
                cnt = jnp.sum(jnp.where(lane == 0, run, 0))
                return lax.cond(cnt > CAP_TRIG, drain_all, lambda x: x, cnt)

            cnt = lax.fori_loop(0, n_chunks, chunk_body, jnp.int32(0))
            drain_all(cnt)
            pltpu.sync_copy(acc.at[pl.ds(0, R)], out_hbm.at[pl.ds(lo, R)])

    f = pl.kernel(
        body,
        out_type=jax.ShapeDtypeStruct((n_pad, bf), jnp.float32),
        mesh=_sc_mesh(),
        compiler_params=pltpu.CompilerParams(needs_layout_passes=False),
        scratch_types=[
            pltpu.VMEM((R, bf), jnp.float32),
            pltpu.VMEM((2 * L, bf), jnp.float32),
            pltpu.VMEM((2 * ECH,), jnp.int32),
            pltpu.VMEM((2 * ECH,), jnp.int32),
            pltpu.VMEM((CAP,), jnp.int32),
            pltpu.VMEM((CAP,), jnp.int32),
            pltpu.VMEM((ECH,), jnp.int32),
            pltpu.SemaphoreType.DMA((2,)),
            pltpu.SemaphoreType.DMA((2,)),
        ],
    )
    return f(y, rows, cols)


def kernel(x, edge_index, W1, b1, W2, b2):
    b_sz, n, f = x.shape
    n_pad = NW * R * -(-n // (NW * R))

    rows = edge_index[0].astype(jnp.int32)
    cols = edge_index[1].astype(jnp.int32)
    e = rows.shape[0]
    e_pad = NW * ECH_D * -(-e // (NW * ECH_D))
    if e_pad != e:
        rows = jnp.concatenate(
            [rows, jnp.zeros((e_pad - e,), jnp.int32)])
        cols = jnp.concatenate(
            [cols, jnp.full((e_pad - e,), n, jnp.int32)])

    x_pad = jnp.pad(x, ((0, 0), (0, n_pad - n), (0, 0)))
    b1r = b1.reshape(1, -1)
    b2r = b2.reshape(1, -1)

    parts = _degree_partials(cols, n_pad)
    dis = _tc_dis(parts, n)

    y1 = _tc_scaled_mm(x_pad, W1, dis)
    acc1 = _sc_spmm(y1, rows, cols)
    y2 = _tc_mid(acc1, dis, b1r, W2)
    acc2 = _sc_spmm(y2, rows, cols)
    out = _tc_final(acc2, dis, b2r)
    return out[:, :n, :]
